# Initial kernel scaffold; baseline (speedup 1.0000x reference)
import jax
import jax.numpy as jnp
from jax.experimental import pallas as pl


def kernel(position_feature, id_feature, edge_index, batch, pos_W, pos_b, emb, node_W, node_b, conv_W, conv_b, agg_W, agg_b):
    raise NotImplementedError("write your pallas kernel here")


# speedup vs baseline: 16.1990x; 16.1990x over previous
import functools

import jax
import jax.numpy as jnp
from jax import lax
from jax.experimental import pallas as pl
from jax.experimental.pallas import tpu as pltpu
from jax.experimental.pallas import tpu_sc as plsc

N = 10000
NPAD = 10240
E = 320000
EPAD = 327680
D = 128
NG = 16
BLK = 512
NBLK = NPAD // BLK
ROWS_PER_SUB = NPAD // 16
CHUNK = 128
NCHUNK = EPAD // (32 * CHUNK)
DEGW = 8

_MESH = dict(core_axis_name="c", subcore_axis_name="s")


def _sc_degree(dst3, ones_hbm, zeros_hbm):
    @functools.partial(
        pl.kernel,
        out_type=jax.ShapeDtypeStruct((2, NPAD, DEGW), jnp.float32),
        mesh=plsc.VectorSubcoreMesh(**_MESH),
        scratch_types=[
            pltpu.VMEM((NCHUNK, CHUNK), jnp.int32),
            pltpu.VMEM((CHUNK, DEGW), jnp.float32),
            pltpu.VMEM_SHARED((NPAD, DEGW), jnp.float32),
        ],
    )
    def k(dst_hbm, one_hbm, z_hbm, out_hbm, dst_v, ones_v, acc_sh):
        cid = lax.axis_index("c")
        sid = lax.axis_index("s")
        w = cid * 16 + sid
        pltpu.sync_copy(dst_hbm.at[w], dst_v)
        pltpu.sync_copy(one_hbm, ones_v)
        pltpu.sync_copy(z_hbm, acc_sh.at[pl.ds(sid * ROWS_PER_SUB, ROWS_PER_SUB)])
        plsc.subcore_barrier()

        def body(j, carry):
            pltpu.sync_copy(ones_v, acc_sh.at[dst_v.at[j]], add=True)
            return carry

        lax.fori_loop(0, NCHUNK, body, 0)
        plsc.subcore_barrier()
        sl = pl.ds(sid * ROWS_PER_SUB, ROWS_PER_SUB)
        pltpu.sync_copy(acc_sh.at[sl], out_hbm.at[cid, sl])

    return k(dst3, ones_hbm, zeros_hbm)


def _sc_aggregate(ht, src3, dst3, zeros_hbm):
    @functools.partial(
        pl.kernel,
        out_type=jax.ShapeDtypeStruct((2, NPAD, D), jnp.float32),
        mesh=plsc.VectorSubcoreMesh(**_MESH),
        scratch_types=[
            pltpu.VMEM((NCHUNK, CHUNK), jnp.int32),
            pltpu.VMEM((NCHUNK, CHUNK), jnp.int32),
            pltpu.VMEM((CHUNK, D), jnp.float32),
            pltpu.VMEM_SHARED((NPAD, D), jnp.float32),
            pltpu.SemaphoreType.DMA,
        ],
    )
    def k(h_hbm, src_hbm, dst_hbm, z_hbm, out_hbm, src_v, dst_v, rows_v, acc_sh, sem):
        cid = lax.axis_index("c")
        sid = lax.axis_index("s")
        w = cid * 16 + sid
        pltpu.sync_copy(src_hbm.at[w], src_v)
        pltpu.sync_copy(dst_hbm.at[w], dst_v)
        pltpu.sync_copy(z_hbm, acc_sh.at[pl.ds(sid * ROWS_PER_SUB, ROWS_PER_SUB)])
        plsc.subcore_barrier()

        def body(j, carry):
            pltpu.async_copy(h_hbm.at[src_v.at[j]], rows_v, sem).wait()
            pltpu.sync_copy(rows_v, acc_sh.at[dst_v.at[j]], add=True)
            return carry

        lax.fori_loop(0, NCHUNK, body, 0)
        plsc.subcore_barrier()
        sl = pl.ds(sid * ROWS_PER_SUB, ROWS_PER_SUB)
        pltpu.sync_copy(acc_sh.at[sl], out_hbm.at[cid, sl])

    return k(ht, src3, dst3, zeros_hbm)


def _pool_block(h, bt):
    cols = []
    for g in range(NG):
        m = jnp.where(bt == g, h, -jnp.inf)
        cols.append(jnp.max(m, axis=0))
    return jnp.stack(cols, axis=0)


def _frontend_body(pos_ref, id_ref, bt_ref, posW_ref, posb_ref, emb_ref,
                   nW_ref, nb_ref, node_ref, pool_ref):
    i = pl.program_id(0)
    pos = jnp.maximum(
        jnp.dot(pos_ref[...], posW_ref[...], preferred_element_type=jnp.float32)
        + posb_ref[...], 0.0)
    ids = id_ref[...]
    oh = (ids == lax.broadcasted_iota(jnp.int32, (BLK, 256), 1))
    idf = jnp.maximum(
        jnp.dot(oh.astype(jnp.float32), emb_ref[...],
                preferred_element_type=jnp.float32), 0.0)
    node = jnp.maximum(
        jnp.dot(pos, nW_ref[:D, :], preferred_element_type=jnp.float32)
        + jnp.dot(idf, nW_ref[D:, :], preferred_element_type=jnp.float32)
        + nb_ref[...], 0.0)
    node_ref[...] = node
    p = _pool_block(node, bt_ref[...])

    @pl.when(i == 0)
    def _():
        pool_ref[...] = p

    @pl.when(i > 0)
    def _():
        pool_ref[...] = jnp.maximum(pool_ref[...], p)


def _tc_frontend(pos_p, id3, batch3, posW8, posb, emb, nodeW, nodeb):
    return pl.pallas_call(
        _frontend_body,
        grid=(NBLK,),
        in_specs=[
            pl.BlockSpec((BLK, 8), lambda i: (i, 0)),
            pl.BlockSpec((BLK, 1), lambda i: (i, 0)),
            pl.BlockSpec((BLK, 1), lambda i: (i, 0)),
            pl.BlockSpec((8, D), lambda i: (0, 0)),
            pl.BlockSpec((1, D), lambda i: (0, 0)),
            pl.BlockSpec((256, D), lambda i: (0, 0)),
            pl.BlockSpec((2 * D, D), lambda i: (0, 0)),
            pl.BlockSpec((1, D), lambda i: (0, 0)),
        ],
        out_specs=[
            pl.BlockSpec((BLK, D), lambda i: (i, 0)),
            pl.BlockSpec((NG, D), lambda i: (0, 0)),
        ],
        out_shape=[
            jax.ShapeDtypeStruct((NPAD, D), jnp.float32),
            jax.ShapeDtypeStruct((NG, D), jnp.float32),
        ],
    )(pos_p, id3, batch3, posW8, posb, emb, nodeW, nodeb)


def _prescale_body(degp_ref, node_ref, W_ref, ht_ref, dis_ref):
    deg = degp_ref[0, :, 0:1] + degp_ref[1, :, 0:1] + 1.0
    dis = lax.rsqrt(deg)
    ht = jnp.dot(node_ref[...], W_ref[...], preferred_element_type=jnp.float32)
    ht_ref[...] = ht * dis
    dis_ref[...] = dis


def _tc_prescale(degp, node, W0):
    return pl.pallas_call(
        _prescale_body,
        grid=(NBLK,),
        in_specs=[
            pl.BlockSpec((2, BLK, DEGW), lambda i: (0, i, 0)),
            pl.BlockSpec((BLK, D), lambda i: (i, 0)),
            pl.BlockSpec((D, D), lambda i: (0, 0)),
        ],
        out_specs=[
            pl.BlockSpec((BLK, D), lambda i: (i, 0)),
            pl.BlockSpec((BLK, 1), lambda i: (i, 0)),
        ],
        out_shape=[
            jax.ShapeDtypeStruct((NPAD, D), jnp.float32),
            jax.ShapeDtypeStruct((NPAD, 1), jnp.float32),
        ],
    )(degp, node, W0)


def _combine_body(acc_ref, ht_ref, dis_ref, b_ref, Wn_ref, bt_ref,
                  htn_ref, pool_ref):
    i = pl.program_id(0)
    s = acc_ref[0] + acc_ref[1] + ht_ref[...]
    h = jnp.maximum(dis_ref[...] * s + b_ref[...], 0.0)
    htn_ref[...] = jnp.dot(h, Wn_ref[...],
                           preferred_element_type=jnp.float32) * dis_ref[...]
    p = _pool_block(h, bt_ref[...])

    @pl.when(i == 0)
    def _():
        pool_ref[...] = p

    @pl.when(i > 0)
    def _():
        pool_ref[...] = jnp.maximum(pool_ref[...], p)


def _tc_combine(acc, ht, dis, b, Wn, batch3):
    return pl.pallas_call(
        _combine_body,
        grid=(NBLK,),
        in_specs=[
            pl.BlockSpec((2, BLK, D), lambda i: (0, i, 0)),
            pl.BlockSpec((BLK, D), lambda i: (i, 0)),
            pl.BlockSpec((BLK, 1), lambda i: (i, 0)),
            pl.BlockSpec((1, D), lambda i: (0, 0)),
            pl.BlockSpec((D, D), lambda i: (0, 0)),
            pl.BlockSpec((BLK, 1), lambda i: (i, 0)),
        ],
        out_specs=[
            pl.BlockSpec((BLK, D), lambda i: (i, 0)),
            pl.BlockSpec((NG, D), lambda i: (0, 0)),
        ],
        out_shape=[
            jax.ShapeDtypeStruct((NPAD, D), jnp.float32),
            jax.ShapeDtypeStruct((NG, D), jnp.float32),
        ],
    )(acc, ht, dis, b, Wn, batch3)


def _final_body(acc_ref, ht_ref, dis_ref, b_ref, bt_ref, p0_ref, p1_ref,
                p2_ref, aggW_ref, aggb_ref, lat_ref, pool_scr):
    i = pl.program_id(0)
    s = acc_ref[0] + acc_ref[1] + ht_ref[...]
    h = jnp.maximum(dis_ref[...] * s + b_ref[...], 0.0)
    p = _pool_block(h, bt_ref[...])

    @pl.when(i == 0)
    def _():
        pool_scr[...] = p

    @pl.when(i > 0)
    def _():
        pool_scr[...] = jnp.maximum(pool_scr[...], p)

    @pl.when(i == NBLK - 1)
    def _():
        lat = (jnp.dot(p0_ref[...], aggW_ref[0:D, :], preferred_element_type=jnp.float32)
               + jnp.dot(p1_ref[...], aggW_ref[D:2 * D, :], preferred_element_type=jnp.float32)
               + jnp.dot(p2_ref[...], aggW_ref[2 * D:3 * D, :], preferred_element_type=jnp.float32)
               + jnp.dot(pool_scr[...], aggW_ref[3 * D:4 * D, :], preferred_element_type=jnp.float32)
               + aggb_ref[...])
        lat_ref[...] = lat


def _tc_final(acc, ht, dis, b, batch3, p0, p1, p2, aggW, aggb):
    return pl.pallas_call(
        _final_body,
        grid=(NBLK,),
        in_specs=[
            pl.BlockSpec((2, BLK, D), lambda i: (0, i, 0)),
            pl.BlockSpec((BLK, D), lambda i: (i, 0)),
            pl.BlockSpec((BLK, 1), lambda i: (i, 0)),
            pl.BlockSpec((1, D), lambda i: (0, 0)),
            pl.BlockSpec((BLK, 1), lambda i: (i, 0)),
            pl.BlockSpec((NG, D), lambda i: (0, 0)),
            pl.BlockSpec((NG, D), lambda i: (0, 0)),
            pl.BlockSpec((NG, D), lambda i: (0, 0)),
            pl.BlockSpec((4 * D, D), lambda i: (0, 0)),
            pl.BlockSpec((1, D), lambda i: (0, 0)),
        ],
        out_specs=pl.BlockSpec((NG, D), lambda i: (0, 0)),
        out_shape=jax.ShapeDtypeStruct((NG, D), jnp.float32),
        scratch_shapes=[pltpu.VMEM((NG, D), jnp.float32)],
    )(acc, ht, dis, b, batch3, p0, p1, p2, aggW, aggb)


def kernel(position_feature, id_feature, edge_index, batch, pos_W, pos_b,
           emb, node_W, node_b, conv_W, conv_b, agg_W, agg_b):
    f32 = jnp.float32
    src = edge_index[0].astype(jnp.int32)
    dst = edge_index[1].astype(jnp.int32)
    npe = EPAD - E
    pad_src = (jnp.arange(npe, dtype=jnp.int32) * 13) % N
    pad_dst = N + (jnp.arange(npe, dtype=jnp.int32) % (NPAD - N))
    src3 = jnp.concatenate([src, pad_src]).reshape(32, NCHUNK, CHUNK)
    dst3 = jnp.concatenate([dst, pad_dst]).reshape(32, NCHUNK, CHUNK)

    pos_p = jnp.pad(position_feature.astype(f32), ((0, NPAD - N), (0, 5)))
    posW8 = jnp.pad(pos_W.astype(f32), ((0, 5), (0, 0)))
    id3 = jnp.pad(id_feature.astype(jnp.int32), (0, NPAD - N)).reshape(NPAD, 1)
    batch3 = jnp.pad(batch.astype(jnp.int32), (0, NPAD - N),
                     constant_values=NG).reshape(NPAD, 1)
    posb = pos_b.astype(f32).reshape(1, D)
    nodeb = node_b.astype(f32).reshape(1, D)
    aggb = agg_b.astype(f32).reshape(1, D)

    zeros_d = jnp.zeros((ROWS_PER_SUB, D), f32)
    zeros_g = jnp.zeros((ROWS_PER_SUB, DEGW), f32)
    ones_g = jnp.ones((CHUNK, DEGW), f32)

    degp = _sc_degree(dst3, ones_g, zeros_g)
    node, pool0 = _tc_frontend(pos_p, id3, batch3, posW8, posb,
                               emb.astype(f32), node_W.astype(f32), nodeb)
    ht, dis = _tc_prescale(degp, node, conv_W[0].astype(f32))

    acc = _sc_aggregate(ht, src3, dst3, zeros_d)
    ht, pool1 = _tc_combine(acc, ht, dis, conv_b[0].astype(f32).reshape(1, D),
                            conv_W[1].astype(f32), batch3)
    acc = _sc_aggregate(ht, src3, dst3, zeros_d)
    ht, pool2 = _tc_combine(acc, ht, dis, conv_b[1].astype(f32).reshape(1, D),
                            conv_W[2].astype(f32), batch3)
    acc = _sc_aggregate(ht, src3, dst3, zeros_d)
    latent = _tc_final(acc, ht, dis, conv_b[2].astype(f32).reshape(1, D),
                       batch3, pool0, pool1, pool2,
                       agg_W.astype(f32), aggb)
    return latent
